# Initial kernel scaffold; baseline (speedup 1.0000x reference)
import jax
import jax.numpy as jnp
from jax.experimental import pallas as pl


def kernel(x, edge_index, edge_attr, Wenc, benc, Wee, bee, Wl, bl, Wr, br, We, att, bias, Wres, bres, W1, b1, W2, b2, W3, b3):
    raise NotImplementedError("write your pallas kernel here")


# speedup vs baseline: 1.0000x; 1.0000x over previous
import jax
import jax.numpy as jnp
from jax.experimental import pallas as pl

N = 10000
E = 160000
D = 128
H = 8
C = 16
L = 14
OUT = 1000


def _elu(v):
    return jnp.where(v > 0, v, jnp.exp(jnp.minimum(v, 0.0)) - 1.0)


def _mlp_head_kernel(g_ref, w1_ref, b1_ref, w2_ref, b2_ref, w3_ref, b3_ref, o_ref):
    g = g_ref[...]
    g = _elu(jnp.dot(g, w1_ref[...], preferred_element_type=jnp.float32) + b1_ref[...])
    g = _elu(jnp.dot(g, w2_ref[...], preferred_element_type=jnp.float32) + b2_ref[...])
    logits = jnp.dot(g, w3_ref[...], preferred_element_type=jnp.float32) + b3_ref[...]
    o_ref[...] = jax.nn.softmax(logits, axis=-1)


def _mlp_head(g, W1, b1, W2, b2, W3, b3):
    return pl.pallas_call(
        _mlp_head_kernel,
        out_shape=jax.ShapeDtypeStruct((1, OUT), jnp.float32),
    )(g, W1, b1[None, :], W2, b2[None, :], W3, b3[None, :])


def kernel(x, edge_index, edge_attr, Wenc, benc, Wee, bee, Wl, bl, Wr, br, We, att, bias, Wres, bres, W1, b1, W2, b2, W3, b3):
    src, dst = edge_index[0], edge_index[1]
    n = x.shape[0]
    h = jax.nn.elu(x @ Wenc + benc)
    ee = jax.nn.elu(edge_attr @ Wee + bee)
    loop = jnp.arange(n, dtype=src.dtype)
    src_f = jnp.concatenate([src, loop])
    dst_f = jnp.concatenate([dst, loop])
    ee_f = jnp.concatenate([ee, jnp.broadcast_to(ee.mean(axis=0, keepdims=True), (n, D))], axis=0)
    for i in range(L):
        res = h
        xl = (h @ Wl[i] + bl[i]).reshape(n, H, C)
        xr = (h @ Wr[i] + br[i]).reshape(n, H, C)
        eproj = (ee_f @ We[i]).reshape(-1, H, C)
        m = xl[src_f] + xr[dst_f] + eproj
        m = jax.nn.leaky_relu(m, negative_slope=0.2)
        alpha = (m * att[i][None, :, :]).sum(-1)
        amax = jax.ops.segment_max(alpha, dst_f, num_segments=n)
        ealpha = jnp.exp(alpha - amax[dst_f])
        denom = jax.ops.segment_sum(ealpha, dst_f, num_segments=n)
        alpha = ealpha / (denom[dst_f] + 1e-16)
        msg = xl[src_f] * alpha[:, :, None]
        out = jax.ops.segment_sum(msg, dst_f, num_segments=n).reshape(n, H * C) + bias[i]
        h = jax.nn.elu(out + res @ Wres[i] + bres[i])
    g = h.mean(axis=0, keepdims=True)
    return _mlp_head(g, W1, b1, W2, b2, W3, b3)


# speedup vs baseline: 14.6271x; 14.6269x over previous
import jax
import jax.numpy as jnp
from jax import lax
from jax.experimental import pallas as pl
from jax.experimental.pallas import tpu as pltpu
from jax.experimental.pallas import tpu_sc as plsc

N = 10000
E = 160000
D = 128
H = 8
C = 16
ND = 34
ED = 10
L = 14
OUT = 1000

NC = 2
NS = 16
LANES = 16

N_PAD = 10240
BPT = 314
CHUNK_E = BPT * 16
E_PAD = NC * NS * CHUNK_E
DEN_R = N_PAD // 16
NEG = 0.2


def _elu(v):
    return jnp.where(v > 0, v, jnp.exp(jnp.minimum(v, 0.0)) - 1.0)


def _encode_x_kernel(x_ref, w_ref, b_ref, o_ref):
    o_ref[...] = _elu(
        jnp.dot(x_ref[...], w_ref[...], preferred_element_type=jnp.float32)
        + b_ref[...])


def _encode_x(x_pad, Wenc, benc):
    bn = 1024
    return pl.pallas_call(
        _encode_x_kernel,
        grid=(N_PAD // bn,),
        in_specs=[
            pl.BlockSpec((bn, ND), lambda i: (i, 0)),
            pl.BlockSpec((ND, D), lambda i: (0, 0)),
            pl.BlockSpec((1, D), lambda i: (0, 0)),
        ],
        out_specs=pl.BlockSpec((bn, D), lambda i: (i, 0)),
        out_shape=jax.ShapeDtypeStruct((N_PAD, D), jnp.float32),
    )(x_pad, Wenc, benc[None, :])


def _encode_e_kernel(ea_ref, w_ref, b_ref, ee_ref, sum_ref):
    i = pl.program_id(0)
    bn = ea_ref.shape[0]
    ee = _elu(
        jnp.dot(ea_ref[...], w_ref[...], preferred_element_type=jnp.float32)
        + b_ref[...])
    ee_ref[...] = ee
    rows = i * bn + lax.broadcasted_iota(jnp.int32, (bn, 1), 0)
    masked = jnp.where(rows < E, ee, 0.0)
    part = jnp.sum(masked, axis=0, keepdims=True)

    @pl.when(i == 0)
    def _():
        sum_ref[...] = jnp.zeros_like(sum_ref)

    sum_ref[...] += part


def _encode_e(edge_attr_pad, Wee, bee):
    bn = 1024
    return pl.pallas_call(
        _encode_e_kernel,
        grid=(E_PAD // bn,),
        in_specs=[
            pl.BlockSpec((bn, ED), lambda i: (i, 0)),
            pl.BlockSpec((ED, D), lambda i: (0, 0)),
            pl.BlockSpec((1, D), lambda i: (0, 0)),
        ],
        out_specs=[
            pl.BlockSpec((bn, D), lambda i: (i, 0)),
            pl.BlockSpec((1, D), lambda i: (0, 0)),
        ],
        out_shape=[
            jax.ShapeDtypeStruct((E_PAD, D), jnp.float32),
            jax.ShapeDtypeStruct((1, D), jnp.float32),
        ],
    )(edge_attr_pad, Wee, bee[None, :])


def _eproj_kernel(ee_ref, we_ref, o_ref):
    o_ref[0] = jnp.dot(ee_ref[...], we_ref[0],
                       preferred_element_type=jnp.float32)


def _eproj_all(ee, We):
    bn = 1024
    return pl.pallas_call(
        _eproj_kernel,
        grid=(L, E_PAD // bn),
        in_specs=[
            pl.BlockSpec((bn, D), lambda l, i: (i, 0)),
            pl.BlockSpec((1, D, D), lambda l, i: (l, 0, 0)),
        ],
        out_specs=pl.BlockSpec((1, bn, D), lambda l, i: (l, i, 0)),
        out_shape=jax.ShapeDtypeStruct((L, E_PAD, D), jnp.float32),
    )(ee, We)


def _project_kernel(h_ref, wl_ref, bl_ref, wr_ref, br_ref, ws_ref, bs_ref,
                    we_ref, me_ref, att_ref, g_ref,
                    xl_ref, xr_ref, hres_ref, lnum_ref, lden_ref):
    h = h_ref[...]
    xl = jnp.dot(h, wl_ref[...], preferred_element_type=jnp.float32) + bl_ref[...]
    xr = jnp.dot(h, wr_ref[...], preferred_element_type=jnp.float32) + br_ref[...]
    xl_ref[...] = xl
    xr_ref[...] = xr
    hres_ref[...] = (
        jnp.dot(h, ws_ref[...], preferred_element_type=jnp.float32) + bs_ref[...])
    eloop = jnp.dot(me_ref[...], we_ref[...], preferred_element_type=jnp.float32)
    m = xl + xr + eloop
    m = jnp.maximum(m, 0.0) + NEG * jnp.minimum(m, 0.0)
    s = jnp.dot(m * att_ref[...], g_ref[...], preferred_element_type=jnp.float32)
    w = jnp.exp(s)
    lnum_ref[...] = w * xl
    lden_ref[...] = w


def _project(h, Wl_i, bl_i, Wr_i, br_i, Ws_i, bs_i, We_i, mean_ee, att_i, G):
    bn = 512
    shp = jax.ShapeDtypeStruct((N_PAD, D), jnp.float32)
    return pl.pallas_call(
        _project_kernel,
        grid=(N_PAD // bn,),
        in_specs=[
            pl.BlockSpec((bn, D), lambda i: (i, 0)),
            pl.BlockSpec((D, D), lambda i: (0, 0)),
            pl.BlockSpec((1, D), lambda i: (0, 0)),
            pl.BlockSpec((D, D), lambda i: (0, 0)),
            pl.BlockSpec((1, D), lambda i: (0, 0)),
            pl.BlockSpec((D, D), lambda i: (0, 0)),
            pl.BlockSpec((1, D), lambda i: (0, 0)),
            pl.BlockSpec((D, D), lambda i: (0, 0)),
            pl.BlockSpec((1, D), lambda i: (0, 0)),
            pl.BlockSpec((1, D), lambda i: (0, 0)),
            pl.BlockSpec((D, D), lambda i: (0, 0)),
        ],
        out_specs=[pl.BlockSpec((bn, D), lambda i: (i, 0))] * 5,
        out_shape=[shp] * 5,
    )(h, Wl_i, bl_i[None, :], Wr_i, br_i[None, :], Ws_i, bs_i[None, :],
      We_i, mean_ee, att_i, G)


def _sc_edges_body(xl_hbm, xr_hbm, ep_hbm, src_hbm, dst_hbm, attspl_hbm,
                   zero_hbm, outn_hbm, outd_hbm,
                   src_c, dst_c, bufs, wmsg, wden, attv, accn, accd,
                   sem0, sem1):
    cid = lax.axis_index("c")
    sid = lax.axis_index("s")
    t = cid * NS + sid
    base_e = t * CHUNK_E
    rows_per_sub = N_PAD // NS
    drows_per_sub = DEN_R // NS

    pltpu.sync_copy(src_hbm.at[pl.ds(base_e, CHUNK_E)], src_c)
    pltpu.sync_copy(dst_hbm.at[pl.ds(base_e, CHUNK_E)], dst_c)
    pltpu.sync_copy(attspl_hbm, attv)

    pltpu.sync_copy(zero_hbm.at[pl.ds(sid * rows_per_sub, rows_per_sub)],
                    accn.at[pl.ds(sid * rows_per_sub, rows_per_sub)])
    pltpu.sync_copy(zero_hbm.at[pl.ds(sid * drows_per_sub, drows_per_sub)],
                    accd.at[pl.ds(sid * drows_per_sub, drows_per_sub)])

    lane = jax.lax.iota(jnp.int32, 16)
    zero16 = jnp.zeros((16,), jnp.float32)
    zero16i = jnp.zeros((16,), jnp.int32)
    sems = (sem0, sem1)

    @pl.loop(0, D)
    def _zero(col):
        plsc.store_scatter(wden, [lane, zero16i + col], zero16)

    def issue_block(blk, j):
        s16 = src_c[pl.ds(blk * 16, 16)]
        d16 = dst_c[pl.ds(blk * 16, 16)]
        pltpu.async_copy(xl_hbm.at[s16], bufs.at[j, 0], sems[j])
        pltpu.async_copy(xr_hbm.at[d16], bufs.at[j, 1], sems[j])
        pltpu.async_copy(ep_hbm.at[pl.ds(base_e + blk * 16, 16)],
                         bufs.at[j, 2], sems[j])

    issue_block(0, 0)
    issue_block(1, 1)

    plsc.subcore_barrier()

    @pl.loop(0, BPT, step=2, init_carry=lane * 8)
    def _blocks(b, prev_dcol):
        for j in range(2):
            blk = b + j
            for sl in range(3):
                pltpu.make_async_copy(xl_hbm.at[pl.ds(0, 16)],
                                      bufs.at[j, sl], sems[j]).wait()
            d16 = dst_c[pl.ds(blk * 16, 16)]
            dcol = (d16 & 15) * 8
            @pl.loop(0, H)
            def _clear(h):
                plsc.store_scatter(wden, [lane, prev_dcol + h], zero16)

            prev_dcol = dcol

            @pl.loop(0, H)
            def _heads(h):
                hC = h * C
                saved = []
                acc_h = None
                for c in range(C):
                    fk = zero16i + (hC + c)
                    xlk = plsc.load_gather(bufs.at[j, 0], [lane, fk])
                    xrk = plsc.load_gather(bufs.at[j, 1], [lane, fk])
                    epk = plsc.load_gather(bufs.at[j, 2], [lane, fk])
                    m = xlk + xrk + epk
                    lr = jnp.maximum(m, 0.0) + NEG * jnp.minimum(m, 0.0)
                    attk = plsc.load_gather(attv, [fk])
                    term = lr * attk
                    acc_h = term if c == 0 else acc_h + term
                    saved.append(xlk)
                w = jnp.exp(acc_h)
                for c in range(C):
                    plsc.store_scatter(wmsg, [lane, zero16i + (hC + c)],
                                       saved[c] * w)
                plsc.store_scatter(wden, [lane, dcol + h], w)

            pltpu.sync_copy(wmsg, accn.at[d16], add=True)
            pltpu.sync_copy(wden, accd.at[d16 >> 4], add=True)

            @pl.when(blk + 2 < BPT)
            def _():
                issue_block(blk + 2, j)
        return prev_dcol

    plsc.subcore_barrier()

    pltpu.sync_copy(accn.at[pl.ds(sid * rows_per_sub, rows_per_sub)],
                    outn_hbm.at[cid, pl.ds(sid * rows_per_sub, rows_per_sub)])
    pltpu.sync_copy(accd.at[pl.ds(sid * drows_per_sub, drows_per_sub)],
                    outd_hbm.at[cid, pl.ds(sid * drows_per_sub, drows_per_sub)])


def _sc_edges(xl, xr, ep_i, src_p, dst_p, attspl_i, zero_acc):
    mesh = plsc.VectorSubcoreMesh(core_axis_name="c", subcore_axis_name="s")
    f = pl.kernel(
        _sc_edges_body,
        out_type=[
            jax.ShapeDtypeStruct((NC, N_PAD, D), jnp.float32),
            jax.ShapeDtypeStruct((NC, DEN_R, D), jnp.float32),
        ],
        mesh=mesh,
        compiler_params=pltpu.CompilerParams(needs_layout_passes=False),
        scratch_types=[
            pltpu.VMEM((CHUNK_E,), jnp.int32),
            pltpu.VMEM((CHUNK_E,), jnp.int32),
            pltpu.VMEM((2, 3, 16, D), jnp.float32),
            pltpu.VMEM((16, D), jnp.float32),
            pltpu.VMEM((16, D), jnp.float32),
            pltpu.VMEM((D,), jnp.float32),
            pltpu.VMEM_SHARED((N_PAD, D), jnp.float32),
            pltpu.VMEM_SHARED((DEN_R, D), jnp.float32),
            pltpu.SemaphoreType.DMA,
            pltpu.SemaphoreType.DMA,
        ],
    )
    return f(xl, xr, ep_i, src_p, dst_p, attspl_i, zero_acc)


def _combine_kernel(n_ref, d_ref, lnum_ref, lden_ref, hres_ref, b_ref,
                    gb_ref, h_ref):
    num = n_ref[0] + n_ref[1] + lnum_ref[...]
    den8 = d_ref[0] + d_ref[1]
    den = (jnp.dot(den8, gb_ref[...], preferred_element_type=jnp.float32)
           + lden_ref[...])
    h_ref[...] = _elu(num / (den + 1e-16) + b_ref[...] + hres_ref[...])


def _combine_call(accn, den8, lnum, lden, hres, bias_i, Gb):
    bn = 512
    return pl.pallas_call(
        _combine_kernel,
        grid=(N_PAD // bn,),
        in_specs=[
            pl.BlockSpec((NC, bn, D), lambda i: (0, i, 0)),
            pl.BlockSpec((NC, bn, H), lambda i: (0, i, 0)),
            pl.BlockSpec((bn, D), lambda i: (i, 0)),
            pl.BlockSpec((bn, D), lambda i: (i, 0)),
            pl.BlockSpec((bn, D), lambda i: (i, 0)),
            pl.BlockSpec((1, D), lambda i: (0, 0)),
            pl.BlockSpec((H, D), lambda i: (0, 0)),
        ],
        out_specs=pl.BlockSpec((bn, D), lambda i: (i, 0)),
        out_shape=jax.ShapeDtypeStruct((N_PAD, D), jnp.float32),
    )(accn, den8, lnum, lden, hres, bias_i[None, :], Gb)


def _head_kernel(h_ref, w1_ref, b1_ref, w2_ref, b2_ref, w3_ref, b3_ref, o_ref):
    rows = lax.broadcasted_iota(jnp.int32, (N_PAD, 1), 0)
    hm = jnp.where(rows < N, h_ref[...], 0.0)
    g = jnp.sum(hm, axis=0, keepdims=True) * (1.0 / N)
    g = _elu(jnp.dot(g, w1_ref[...], preferred_element_type=jnp.float32) + b1_ref[...])
    g = _elu(jnp.dot(g, w2_ref[...], preferred_element_type=jnp.float32) + b2_ref[...])
    logits = jnp.dot(g, w3_ref[...], preferred_element_type=jnp.float32) + b3_ref[...]
    z = logits - jnp.max(logits, axis=-1, keepdims=True)
    ez = jnp.exp(z)
    o_ref[...] = ez / jnp.sum(ez, axis=-1, keepdims=True)


def _head(h, W1, b1, W2, b2, W3, b3):
    return pl.pallas_call(
        _head_kernel,
        out_shape=jax.ShapeDtypeStruct((1, OUT), jnp.float32),
    )(h, W1, b1[None, :], W2, b2[None, :], W3, b3[None, :])


def kernel(x, edge_index, edge_attr, Wenc, benc, Wee, bee, Wl, bl, Wr, br,
           We, att, bias, Wres, bres, W1, b1, W2, b2, W3, b3):
    src, dst = edge_index[0], edge_index[1]

    pad_e = E_PAD - E
    src_p = jnp.concatenate([src, jnp.full((pad_e,), N, jnp.int32)])
    dst_p = jnp.concatenate([dst, jnp.full((pad_e,), N, jnp.int32)])
    x_pad = jnp.zeros((N_PAD, ND), jnp.float32).at[:N].set(x)
    ea_pad = jnp.zeros((E_PAD, ED), jnp.float32).at[:E].set(edge_attr)
    kk = jnp.arange(D)
    G = (kk[:, None] // C == kk[None, :] // C).astype(jnp.float32)
    Gb = (jnp.arange(H)[:, None] == kk[None, :] // C).astype(jnp.float32)
    att_rows = att.reshape(L, 1, D)
    attflat = att.reshape(L, D)
    zero_acc = jnp.zeros((N_PAD, D), jnp.float32)

    h = _encode_x(x_pad, Wenc, benc)
    ee, ee_sum = _encode_e(ea_pad, Wee, bee)
    mean_ee = ee_sum * (1.0 / E)

    ep_all = _eproj_all(ee, We)

    for i in range(L):
        xl, xr, hres, lnum, lden = _project(
            h, Wl[i], bl[i], Wr[i], br[i], Wres[i], bres[i], We[i],
            mean_ee, att_rows[i], G)
        accn, accd = _sc_edges(xl, xr, ep_all[i], src_p, dst_p, attflat[i],
                               zero_acc)
        den8 = accd.reshape(NC, N_PAD, H)
        h = _combine_call(accn, den8, lnum, lden, hres, bias[i], Gb)

    return _head(h, W1, b1, W2, b2, W3, b3)


# speedup vs baseline: 15.2454x; 1.0423x over previous
import jax
import jax.numpy as jnp
from jax import lax
from jax.experimental import pallas as pl
from jax.experimental.pallas import tpu as pltpu
from jax.experimental.pallas import tpu_sc as plsc

N = 10000
E = 160000
D = 128
H = 8
C = 16
ND = 34
ED = 10
L = 14
OUT = 1000

NC = 2
NS = 16
LANES = 16

N_PAD = 10240
BPT = 316
CHUNK_E = BPT * 16
E_PAD = NC * NS * CHUNK_E
DEN_R = N_PAD // 16
NEG = 0.2


def _elu(v):
    return jnp.where(v > 0, v, jnp.exp(jnp.minimum(v, 0.0)) - 1.0)


def _encode_x_kernel(x_ref, w_ref, b_ref, o_ref):
    o_ref[...] = _elu(
        jnp.dot(x_ref[...], w_ref[...], preferred_element_type=jnp.float32)
        + b_ref[...])


def _encode_x(x_pad, Wenc, benc):
    bn = 1024
    return pl.pallas_call(
        _encode_x_kernel,
        grid=(N_PAD // bn,),
        in_specs=[
            pl.BlockSpec((bn, ND), lambda i: (i, 0)),
            pl.BlockSpec((ND, D), lambda i: (0, 0)),
            pl.BlockSpec((1, D), lambda i: (0, 0)),
        ],
        out_specs=pl.BlockSpec((bn, D), lambda i: (i, 0)),
        out_shape=jax.ShapeDtypeStruct((N_PAD, D), jnp.float32),
    )(x_pad, Wenc, benc[None, :])


def _encode_e_kernel(ea_ref, w_ref, b_ref, ee_ref, sum_ref):
    i = pl.program_id(0)
    bn = ea_ref.shape[0]
    ee = _elu(
        jnp.dot(ea_ref[...], w_ref[...], preferred_element_type=jnp.float32)
        + b_ref[...])
    ee_ref[...] = ee
    rows = i * bn + lax.broadcasted_iota(jnp.int32, (bn, 1), 0)
    masked = jnp.where(rows < E, ee, 0.0)
    part = jnp.sum(masked, axis=0, keepdims=True)

    @pl.when(i == 0)
    def _():
        sum_ref[...] = jnp.zeros_like(sum_ref)

    sum_ref[...] += part


def _encode_e(edge_attr_pad, Wee, bee):
    bn = 1024
    return pl.pallas_call(
        _encode_e_kernel,
        grid=(E_PAD // bn,),
        in_specs=[
            pl.BlockSpec((bn, ED), lambda i: (i, 0)),
            pl.BlockSpec((ED, D), lambda i: (0, 0)),
            pl.BlockSpec((1, D), lambda i: (0, 0)),
        ],
        out_specs=[
            pl.BlockSpec((bn, D), lambda i: (i, 0)),
            pl.BlockSpec((1, D), lambda i: (0, 0)),
        ],
        out_shape=[
            jax.ShapeDtypeStruct((E_PAD, D), jnp.float32),
            jax.ShapeDtypeStruct((1, D), jnp.float32),
        ],
    )(edge_attr_pad, Wee, bee[None, :])


def _eproj_kernel(ee_ref, we_ref, o_ref):
    o_ref[0] = jnp.dot(ee_ref[...], we_ref[0],
                       preferred_element_type=jnp.float32)


def _eproj_all(ee, We):
    bn = 1024
    return pl.pallas_call(
        _eproj_kernel,
        grid=(L, E_PAD // bn),
        in_specs=[
            pl.BlockSpec((bn, D), lambda l, i: (i, 0)),
            pl.BlockSpec((1, D, D), lambda l, i: (l, 0, 0)),
        ],
        out_specs=pl.BlockSpec((1, bn, D), lambda l, i: (l, i, 0)),
        out_shape=jax.ShapeDtypeStruct((L, E_PAD, D), jnp.float32),
    )(ee, We)


def _project_kernel(h_ref, wl_ref, bl_ref, wr_ref, br_ref, ws_ref, bs_ref,
                    we_ref, me_ref, att_ref, g_ref,
                    xl_ref, xr_ref, hres_ref, lnum_ref, lden_ref):
    h = h_ref[...]
    xl = jnp.dot(h, wl_ref[...], preferred_element_type=jnp.float32) + bl_ref[...]
    xr = jnp.dot(h, wr_ref[...], preferred_element_type=jnp.float32) + br_ref[...]
    xl_ref[...] = xl
    xr_ref[...] = xr
    hres_ref[...] = (
        jnp.dot(h, ws_ref[...], preferred_element_type=jnp.float32) + bs_ref[...])
    eloop = jnp.dot(me_ref[...], we_ref[...], preferred_element_type=jnp.float32)
    m = xl + xr + eloop
    m = jnp.maximum(m, 0.0) + NEG * jnp.minimum(m, 0.0)
    s = jnp.dot(m * att_ref[...], g_ref[...], preferred_element_type=jnp.float32)
    w = jnp.exp(s)
    lnum_ref[...] = w * xl
    lden_ref[...] = w


def _project(h, Wl_i, bl_i, Wr_i, br_i, Ws_i, bs_i, We_i, mean_ee, att_i, G):
    bn = 512
    shp = jax.ShapeDtypeStruct((N_PAD, D), jnp.float32)
    return pl.pallas_call(
        _project_kernel,
        grid=(N_PAD // bn,),
        in_specs=[
            pl.BlockSpec((bn, D), lambda i: (i, 0)),
            pl.BlockSpec((D, D), lambda i: (0, 0)),
            pl.BlockSpec((1, D), lambda i: (0, 0)),
            pl.BlockSpec((D, D), lambda i: (0, 0)),
            pl.BlockSpec((1, D), lambda i: (0, 0)),
            pl.BlockSpec((D, D), lambda i: (0, 0)),
            pl.BlockSpec((1, D), lambda i: (0, 0)),
            pl.BlockSpec((D, D), lambda i: (0, 0)),
            pl.BlockSpec((1, D), lambda i: (0, 0)),
            pl.BlockSpec((1, D), lambda i: (0, 0)),
            pl.BlockSpec((D, D), lambda i: (0, 0)),
        ],
        out_specs=[pl.BlockSpec((bn, D), lambda i: (i, 0))] * 5,
        out_shape=[shp] * 5,
    )(h, Wl_i, bl_i[None, :], Wr_i, br_i[None, :], Ws_i, bs_i[None, :],
      We_i, mean_ee, att_i, G)


NBUF = 4
SBUF = 2


def _sc_edges_body(xl_hbm, xr_hbm, ep_hbm, src_hbm, dst_hbm, attspl_hbm,
                   zero_hbm, outn_hbm, outd_hbm,
                   src_c, dst_c, bufs, wmsg, wden, attv, accn, accd,
                   isems, nsems, dsems):
    cid = lax.axis_index("c")
    sid = lax.axis_index("s")
    t = cid * NS + sid
    base_e = t * CHUNK_E
    rows_per_sub = N_PAD // NS
    drows_per_sub = DEN_R // NS

    pltpu.sync_copy(src_hbm.at[pl.ds(base_e, CHUNK_E)], src_c)
    pltpu.sync_copy(dst_hbm.at[pl.ds(base_e, CHUNK_E)], dst_c)
    pltpu.sync_copy(attspl_hbm, attv)

    pltpu.sync_copy(zero_hbm.at[pl.ds(sid * rows_per_sub, rows_per_sub)],
                    accn.at[pl.ds(sid * rows_per_sub, rows_per_sub)])
    pltpu.sync_copy(zero_hbm.at[pl.ds(sid * drows_per_sub, drows_per_sub)],
                    accd.at[pl.ds(sid * drows_per_sub, drows_per_sub)])

    lane = jax.lax.iota(jnp.int32, 16)
    zero16 = jnp.zeros((16,), jnp.float32)
    zero16i = jnp.zeros((16,), jnp.int32)

    for sj in range(SBUF):
        @pl.loop(0, D)
        def _zero(col, sj=sj):
            plsc.store_scatter(wden.at[sj], [lane, zero16i + col], zero16)

    def issue_block(blk, j):
        s16 = src_c[pl.ds(blk * 16, 16)]
        d16 = dst_c[pl.ds(blk * 16, 16)]
        pltpu.async_copy(xl_hbm.at[s16], bufs.at[j, 0], isems.at[j])
        pltpu.async_copy(xr_hbm.at[d16], bufs.at[j, 1], isems.at[j])
        pltpu.async_copy(ep_hbm.at[pl.ds(base_e + blk * 16, 16)],
                         bufs.at[j, 2], isems.at[j])

    for j in range(NBUF):
        issue_block(j, j)

    plsc.subcore_barrier()

    @pl.loop(0, BPT, step=NBUF, init_carry=tuple(lane * 8 for _ in range(SBUF)))
    def _blocks(b, prev_dcols):
        prev_dcols = list(prev_dcols)
        for j in range(NBUF):
            sj = j % SBUF
            blk = b + j
            for sl in range(3):
                pltpu.make_async_copy(xl_hbm.at[pl.ds(0, 16)],
                                      bufs.at[j, sl], isems.at[j]).wait()
            d16 = dst_c[pl.ds(blk * 16, 16)]
            dcol = (d16 & 15) * 8

            @pl.when(blk >= SBUF)
            def _():
                pltpu.make_async_copy(wmsg.at[sj], accn.at[d16],
                                      nsems.at[sj]).wait()
                pltpu.make_async_copy(wden.at[sj], accd.at[d16 >> 4],
                                      dsems.at[sj]).wait()

            @pl.loop(0, H)
            def _clear(h):
                plsc.store_scatter(wden.at[sj], [lane, prev_dcols[sj] + h],
                                   zero16)

            prev_dcols[sj] = dcol

            @pl.loop(0, H)
            def _heads(h):
                hC = h * C
                saved = []
                acc_h = None
                for c in range(C):
                    fk = zero16i + (hC + c)
                    xlk = plsc.load_gather(bufs.at[j, 0], [lane, fk])
                    xrk = plsc.load_gather(bufs.at[j, 1], [lane, fk])
                    epk = plsc.load_gather(bufs.at[j, 2], [lane, fk])
                    m = xlk + xrk + epk
                    lr = jnp.maximum(m, 0.0) + NEG * jnp.minimum(m, 0.0)
                    attk = plsc.load_gather(attv, [fk])
                    term = lr * attk
                    acc_h = term if c == 0 else acc_h + term
                    saved.append(xlk)
                w = jnp.exp(acc_h)
                for c in range(C):
                    plsc.store_scatter(wmsg.at[sj], [lane, zero16i + (hC + c)],
                                       saved[c] * w)
                plsc.store_scatter(wden.at[sj], [lane, dcol + h], w)

            pltpu.async_copy(wmsg.at[sj], accn.at[d16], nsems.at[sj], add=True)
            pltpu.async_copy(wden.at[sj], accd.at[d16 >> 4], dsems.at[sj],
                             add=True)

            @pl.when(blk + NBUF < BPT)
            def _():
                issue_block(blk + NBUF, j)
        return tuple(prev_dcols)

    for sj in range(SBUF):
        pltpu.make_async_copy(wmsg.at[sj], accn.at[pl.ds(0, 16)],
                              nsems.at[sj]).wait()
        pltpu.make_async_copy(wden.at[sj], accd.at[pl.ds(0, 16)],
                              dsems.at[sj]).wait()

    plsc.subcore_barrier()

    pltpu.sync_copy(accn.at[pl.ds(sid * rows_per_sub, rows_per_sub)],
                    outn_hbm.at[cid, pl.ds(sid * rows_per_sub, rows_per_sub)])
    pltpu.sync_copy(accd.at[pl.ds(sid * drows_per_sub, drows_per_sub)],
                    outd_hbm.at[cid, pl.ds(sid * drows_per_sub, drows_per_sub)])


def _sc_edges(xl, xr, ep_i, src_p, dst_p, attspl_i, zero_acc):
    mesh = plsc.VectorSubcoreMesh(core_axis_name="c", subcore_axis_name="s")
    f = pl.kernel(
        _sc_edges_body,
        out_type=[
            jax.ShapeDtypeStruct((NC, N_PAD, D), jnp.float32),
            jax.ShapeDtypeStruct((NC, DEN_R, D), jnp.float32),
        ],
        mesh=mesh,
        compiler_params=pltpu.CompilerParams(needs_layout_passes=False),
        scratch_types=[
            pltpu.VMEM((CHUNK_E,), jnp.int32),
            pltpu.VMEM((CHUNK_E,), jnp.int32),
            pltpu.VMEM((NBUF, 3, 16, D), jnp.float32),
            pltpu.VMEM((SBUF, 16, D), jnp.float32),
            pltpu.VMEM((SBUF, 16, D), jnp.float32),
            pltpu.VMEM((D,), jnp.float32),
            pltpu.VMEM_SHARED((N_PAD, D), jnp.float32),
            pltpu.VMEM_SHARED((DEN_R, D), jnp.float32),
            pltpu.SemaphoreType.DMA((NBUF,)),
            pltpu.SemaphoreType.DMA((SBUF,)),
            pltpu.SemaphoreType.DMA((SBUF,)),
        ],
    )
    return f(xl, xr, ep_i, src_p, dst_p, attspl_i, zero_acc)


def _combine_kernel(n_ref, d_ref, lnum_ref, lden_ref, hres_ref, b_ref,
                    gb_ref, h_ref):
    num = n_ref[0] + n_ref[1] + lnum_ref[...]
    den8 = d_ref[0] + d_ref[1]
    den = (jnp.dot(den8, gb_ref[...], preferred_element_type=jnp.float32)
           + lden_ref[...])
    h_ref[...] = _elu(num / (den + 1e-16) + b_ref[...] + hres_ref[...])


def _combine_call(accn, den8, lnum, lden, hres, bias_i, Gb):
    bn = 512
    return pl.pallas_call(
        _combine_kernel,
        grid=(N_PAD // bn,),
        in_specs=[
            pl.BlockSpec((NC, bn, D), lambda i: (0, i, 0)),
            pl.BlockSpec((NC, bn, H), lambda i: (0, i, 0)),
            pl.BlockSpec((bn, D), lambda i: (i, 0)),
            pl.BlockSpec((bn, D), lambda i: (i, 0)),
            pl.BlockSpec((bn, D), lambda i: (i, 0)),
            pl.BlockSpec((1, D), lambda i: (0, 0)),
            pl.BlockSpec((H, D), lambda i: (0, 0)),
        ],
        out_specs=pl.BlockSpec((bn, D), lambda i: (i, 0)),
        out_shape=jax.ShapeDtypeStruct((N_PAD, D), jnp.float32),
    )(accn, den8, lnum, lden, hres, bias_i[None, :], Gb)


def _head_kernel(h_ref, w1_ref, b1_ref, w2_ref, b2_ref, w3_ref, b3_ref, o_ref):
    rows = lax.broadcasted_iota(jnp.int32, (N_PAD, 1), 0)
    hm = jnp.where(rows < N, h_ref[...], 0.0)
    g = jnp.sum(hm, axis=0, keepdims=True) * (1.0 / N)
    g = _elu(jnp.dot(g, w1_ref[...], preferred_element_type=jnp.float32) + b1_ref[...])
    g = _elu(jnp.dot(g, w2_ref[...], preferred_element_type=jnp.float32) + b2_ref[...])
    logits = jnp.dot(g, w3_ref[...], preferred_element_type=jnp.float32) + b3_ref[...]
    z = logits - jnp.max(logits, axis=-1, keepdims=True)
    ez = jnp.exp(z)
    o_ref[...] = ez / jnp.sum(ez, axis=-1, keepdims=True)


def _head(h, W1, b1, W2, b2, W3, b3):
    return pl.pallas_call(
        _head_kernel,
        out_shape=jax.ShapeDtypeStruct((1, OUT), jnp.float32),
    )(h, W1, b1[None, :], W2, b2[None, :], W3, b3[None, :])


def kernel(x, edge_index, edge_attr, Wenc, benc, Wee, bee, Wl, bl, Wr, br,
           We, att, bias, Wres, bres, W1, b1, W2, b2, W3, b3):
    src, dst = edge_index[0], edge_index[1]

    pad_e = E_PAD - E
    src_p = jnp.concatenate([src, jnp.full((pad_e,), N, jnp.int32)])
    dst_p = jnp.concatenate([dst, jnp.full((pad_e,), N, jnp.int32)])
    x_pad = jnp.zeros((N_PAD, ND), jnp.float32).at[:N].set(x)
    ea_pad = jnp.zeros((E_PAD, ED), jnp.float32).at[:E].set(edge_attr)
    kk = jnp.arange(D)
    G = (kk[:, None] // C == kk[None, :] // C).astype(jnp.float32)
    Gb = (jnp.arange(H)[:, None] == kk[None, :] // C).astype(jnp.float32)
    att_rows = att.reshape(L, 1, D)
    attflat = att.reshape(L, D)
    zero_acc = jnp.zeros((N_PAD, D), jnp.float32)

    h = _encode_x(x_pad, Wenc, benc)
    ee, ee_sum = _encode_e(ea_pad, Wee, bee)
    mean_ee = ee_sum * (1.0 / E)

    ep_all = _eproj_all(ee, We)

    for i in range(L):
        xl, xr, hres, lnum, lden = _project(
            h, Wl[i], bl[i], Wr[i], br[i], Wres[i], bres[i], We[i],
            mean_ee, att_rows[i], G)
        accn, accd = _sc_edges(xl, xr, ep_all[i], src_p, dst_p, attflat[i],
                               zero_acc)
        den8 = accd.reshape(NC, N_PAD, H)
        h = _combine_call(accn, den8, lnum, lden, hres, bias[i], Gb)

    return _head(h, W1, b1, W2, b2, W3, b3)


# speedup vs baseline: 42.6631x; 2.7984x over previous
import jax
import jax.numpy as jnp
from jax import lax
from jax.experimental import pallas as pl
from jax.experimental.pallas import tpu as pltpu
from jax.experimental.pallas import tpu_sc as plsc

N = 10000
E = 160000
D = 128
H = 8
C = 16
ND = 34
ED = 10
L = 14
OUT = 1000

NC = 2
NS = 16
LANES = 16

N_PAD = 10240
BPT = 316
CHUNK_E = BPT * 16
E_PAD = NC * NS * CHUNK_E
DEN_R = N_PAD // 16
NEG = 0.2


def _elu(v):
    return jnp.where(v > 0, v, jnp.exp(jnp.minimum(v, 0.0)) - 1.0)


def _encode_x_kernel(x_ref, w_ref, b_ref, o_ref):
    o_ref[...] = _elu(
        jnp.dot(x_ref[...], w_ref[...], preferred_element_type=jnp.float32)
        + b_ref[...])


def _encode_x(x_pad, Wenc, benc):
    bn = 1024
    return pl.pallas_call(
        _encode_x_kernel,
        grid=(N_PAD // bn,),
        in_specs=[
            pl.BlockSpec((bn, ND), lambda i: (i, 0)),
            pl.BlockSpec((ND, D), lambda i: (0, 0)),
            pl.BlockSpec((1, D), lambda i: (0, 0)),
        ],
        out_specs=pl.BlockSpec((bn, D), lambda i: (i, 0)),
        out_shape=jax.ShapeDtypeStruct((N_PAD, D), jnp.float32),
    )(x_pad, Wenc, benc[None, :])


def _encode_e_kernel(ea_ref, w_ref, b_ref, ee_ref, sum_ref):
    i = pl.program_id(0)
    bn = ea_ref.shape[0]
    ee = _elu(
        jnp.dot(ea_ref[...], w_ref[...], preferred_element_type=jnp.float32)
        + b_ref[...])
    ee_ref[...] = ee
    rows = i * bn + lax.broadcasted_iota(jnp.int32, (bn, 1), 0)
    masked = jnp.where(rows < E, ee, 0.0)
    part = jnp.sum(masked, axis=0, keepdims=True)

    @pl.when(i == 0)
    def _():
        sum_ref[...] = jnp.zeros_like(sum_ref)

    sum_ref[...] += part


def _encode_e(edge_attr_pad, Wee, bee):
    bn = 1024
    return pl.pallas_call(
        _encode_e_kernel,
        grid=(E_PAD // bn,),
        in_specs=[
            pl.BlockSpec((bn, ED), lambda i: (i, 0)),
            pl.BlockSpec((ED, D), lambda i: (0, 0)),
            pl.BlockSpec((1, D), lambda i: (0, 0)),
        ],
        out_specs=[
            pl.BlockSpec((bn, D), lambda i: (i, 0)),
            pl.BlockSpec((1, D), lambda i: (0, 0)),
        ],
        out_shape=[
            jax.ShapeDtypeStruct((E_PAD, D), jnp.float32),
            jax.ShapeDtypeStruct((1, D), jnp.float32),
        ],
    )(edge_attr_pad, Wee, bee[None, :])


def _eproj_kernel(ee_ref, we_ref, o_ref):
    o_ref[0] = jnp.dot(ee_ref[...], we_ref[0],
                       preferred_element_type=jnp.float32)


def _eproj_all(ee, We):
    bn = 1024
    return pl.pallas_call(
        _eproj_kernel,
        grid=(L, E_PAD // bn),
        in_specs=[
            pl.BlockSpec((bn, D), lambda l, i: (i, 0)),
            pl.BlockSpec((1, D, D), lambda l, i: (l, 0, 0)),
        ],
        out_specs=pl.BlockSpec((1, bn, D), lambda l, i: (l, i, 0)),
        out_shape=jax.ShapeDtypeStruct((L, E_PAD, D), jnp.float32),
    )(ee, We)


def _project_kernel(h_ref, wl_ref, bl_ref, wr_ref, br_ref, ws_ref, bs_ref,
                    we_ref, me_ref, att_ref, g_ref,
                    xl_ref, xr_ref, hres_ref, lnum_ref, lden_ref):
    h = h_ref[...]
    xl = jnp.dot(h, wl_ref[...], preferred_element_type=jnp.float32) + bl_ref[...]
    xr = jnp.dot(h, wr_ref[...], preferred_element_type=jnp.float32) + br_ref[...]
    xl_ref[...] = xl
    xr_ref[...] = xr
    hres_ref[...] = (
        jnp.dot(h, ws_ref[...], preferred_element_type=jnp.float32) + bs_ref[...])
    eloop = jnp.dot(me_ref[...], we_ref[...], preferred_element_type=jnp.float32)
    m = xl + xr + eloop
    m = jnp.maximum(m, 0.0) + NEG * jnp.minimum(m, 0.0)
    s = jnp.dot(m * att_ref[...], g_ref[...], preferred_element_type=jnp.float32)
    w = jnp.exp(s)
    lnum_ref[...] = w * xl
    lden_ref[...] = w


def _project(h, Wl_i, bl_i, Wr_i, br_i, Ws_i, bs_i, We_i, mean_ee, att_i, G):
    bn = 512
    shp = jax.ShapeDtypeStruct((N_PAD, D), jnp.float32)
    return pl.pallas_call(
        _project_kernel,
        grid=(N_PAD // bn,),
        in_specs=[
            pl.BlockSpec((bn, D), lambda i: (i, 0)),
            pl.BlockSpec((D, D), lambda i: (0, 0)),
            pl.BlockSpec((1, D), lambda i: (0, 0)),
            pl.BlockSpec((D, D), lambda i: (0, 0)),
            pl.BlockSpec((1, D), lambda i: (0, 0)),
            pl.BlockSpec((D, D), lambda i: (0, 0)),
            pl.BlockSpec((1, D), lambda i: (0, 0)),
            pl.BlockSpec((D, D), lambda i: (0, 0)),
            pl.BlockSpec((1, D), lambda i: (0, 0)),
            pl.BlockSpec((1, D), lambda i: (0, 0)),
            pl.BlockSpec((D, D), lambda i: (0, 0)),
        ],
        out_specs=[pl.BlockSpec((bn, D), lambda i: (i, 0))] * 5,
        out_shape=[shp] * 5,
    )(h, Wl_i, bl_i[None, :], Wr_i, br_i[None, :], Ws_i, bs_i[None, :],
      We_i, mean_ee, att_i, G)


NBUF = 2
SBUF = 2


def _sc_edges_body(xl_hbm, xr_hbm, ep_hbm, src_hbm, dst_hbm, attspl_hbm,
                   zero_hbm, outn_hbm, outd_hbm,
                   src_c, dst_c, bufs, wmsg, wden, attv, accn, accd,
                   isems, nsems, dsems):
    cid = lax.axis_index("c")
    sid = lax.axis_index("s")
    t = cid * NS + sid
    base_e = t * CHUNK_E
    rows_per_sub = N_PAD // NS
    drows_per_sub = DEN_R // NS

    pltpu.sync_copy(src_hbm.at[pl.ds(base_e, CHUNK_E)], src_c)
    pltpu.sync_copy(dst_hbm.at[pl.ds(base_e, CHUNK_E)], dst_c)
    pltpu.sync_copy(attspl_hbm, attv)

    pltpu.sync_copy(zero_hbm.at[pl.ds(sid * rows_per_sub, rows_per_sub)],
                    accn.at[pl.ds(sid * rows_per_sub, rows_per_sub)])
    pltpu.sync_copy(zero_hbm.at[pl.ds(sid * drows_per_sub, drows_per_sub)],
                    accd.at[pl.ds(sid * drows_per_sub, drows_per_sub)])

    lane = jax.lax.iota(jnp.int32, 16)
    zero16 = jnp.zeros((16,), jnp.float32)
    zero16i = jnp.zeros((16,), jnp.int32)

    for sj in range(SBUF):
        @pl.loop(0, D)
        def _zero(col, sj=sj):
            plsc.store_scatter(wden.at[sj], [lane, zero16i + col], zero16)

    def issue_block(blk, j):
        s16 = src_c[pl.ds(blk * 16, 16)]
        d16 = dst_c[pl.ds(blk * 16, 16)]
        pltpu.async_copy(xl_hbm.at[s16], bufs.at[j, 0], isems.at[j])
        pltpu.async_copy(xr_hbm.at[d16], bufs.at[j, 1], isems.at[j])
        pltpu.async_copy(ep_hbm.at[pl.ds(base_e + blk * 16, 16)],
                         bufs.at[j, 2], isems.at[j])

    for j in range(NBUF):
        issue_block(j, j)

    plsc.subcore_barrier()

    @pl.loop(0, BPT, step=NBUF, init_carry=tuple(lane * 8 for _ in range(SBUF)))
    def _blocks(b, prev_dcols):
        prev_dcols = list(prev_dcols)
        for j in range(NBUF):
            sj = j % SBUF
            blk = b + j
            for sl in range(3):
                pltpu.make_async_copy(xl_hbm.at[pl.ds(0, 16)],
                                      bufs.at[j, sl], isems.at[j]).wait()
            d16 = dst_c[pl.ds(blk * 16, 16)]
            dcol = (d16 & 15) * 8

            @pl.when(blk >= SBUF)
            def _():
                pltpu.make_async_copy(wmsg.at[sj], accn.at[d16],
                                      nsems.at[sj]).wait()
                pltpu.make_async_copy(wden.at[sj], accd.at[d16 >> 4],
                                      dsems.at[sj]).wait()

            @pl.loop(0, H)
            def _clear(h):
                plsc.store_scatter(wden.at[sj], [lane, prev_dcols[sj] + h],
                                   zero16)

            prev_dcols[sj] = dcol

            @pl.loop(0, H)
            def _heads(h):
                hC = h * C
                saved = []
                acc_h = None
                for c in range(C):
                    fk = ((lane + c) & 15) + hC
                    xlk = plsc.load_gather(bufs.at[j, 0], [lane, fk])
                    xrk = plsc.load_gather(bufs.at[j, 1], [lane, fk])
                    epk = plsc.load_gather(bufs.at[j, 2], [lane, fk])
                    m = xlk + xrk + epk
                    lr = jnp.maximum(m, NEG * m)
                    attk = plsc.load_gather(attv, [fk])
                    term = lr * attk
                    acc_h = term if c == 0 else acc_h + term
                    saved.append(xlk)
                w = jnp.exp(acc_h)
                for c in range(C):
                    fk = ((lane + c) & 15) + hC
                    plsc.store_scatter(wmsg.at[sj], [lane, fk], saved[c] * w)
                plsc.store_scatter(wden.at[sj], [lane, dcol + h], w)

            pltpu.async_copy(wmsg.at[sj], accn.at[d16], nsems.at[sj], add=True)
            pltpu.async_copy(wden.at[sj], accd.at[d16 >> 4], dsems.at[sj],
                             add=True)

            @pl.when(blk + NBUF < BPT)
            def _():
                issue_block(blk + NBUF, j)
        return tuple(prev_dcols)

    for sj in range(SBUF):
        pltpu.make_async_copy(wmsg.at[sj], accn.at[pl.ds(0, 16)],
                              nsems.at[sj]).wait()
        pltpu.make_async_copy(wden.at[sj], accd.at[pl.ds(0, 16)],
                              dsems.at[sj]).wait()

    plsc.subcore_barrier()

    pltpu.sync_copy(accn.at[pl.ds(sid * rows_per_sub, rows_per_sub)],
                    outn_hbm.at[cid, pl.ds(sid * rows_per_sub, rows_per_sub)])
    pltpu.sync_copy(accd.at[pl.ds(sid * drows_per_sub, drows_per_sub)],
                    outd_hbm.at[cid, pl.ds(sid * drows_per_sub, drows_per_sub)])


def _sc_edges(xl, xr, ep_i, src_p, dst_p, attspl_i, zero_acc):
    mesh = plsc.VectorSubcoreMesh(core_axis_name="c", subcore_axis_name="s")
    f = pl.kernel(
        _sc_edges_body,
        out_type=[
            jax.ShapeDtypeStruct((NC, N_PAD, D), jnp.float32),
            jax.ShapeDtypeStruct((NC, DEN_R, D), jnp.float32),
        ],
        mesh=mesh,
        compiler_params=pltpu.CompilerParams(needs_layout_passes=False),
        scratch_types=[
            pltpu.VMEM((CHUNK_E,), jnp.int32),
            pltpu.VMEM((CHUNK_E,), jnp.int32),
            pltpu.VMEM((NBUF, 3, 16, D), jnp.float32),
            pltpu.VMEM((SBUF, 16, D), jnp.float32),
            pltpu.VMEM((SBUF, 16, D), jnp.float32),
            pltpu.VMEM((D,), jnp.float32),
            pltpu.VMEM_SHARED((N_PAD, D), jnp.float32),
            pltpu.VMEM_SHARED((DEN_R, D), jnp.float32),
            pltpu.SemaphoreType.DMA((NBUF,)),
            pltpu.SemaphoreType.DMA((SBUF,)),
            pltpu.SemaphoreType.DMA((SBUF,)),
        ],
    )
    return f(xl, xr, ep_i, src_p, dst_p, attspl_i, zero_acc)


def _combine_kernel(n_ref, d_ref, lnum_ref, lden_ref, hres_ref, b_ref,
                    gb_ref, h_ref):
    num = n_ref[0] + n_ref[1] + lnum_ref[...]
    den8 = d_ref[0] + d_ref[1]
    den = (jnp.dot(den8, gb_ref[...], preferred_element_type=jnp.float32)
           + lden_ref[...])
    h_ref[...] = _elu(num / (den + 1e-16) + b_ref[...] + hres_ref[...])


def _combine_call(accn, den8, lnum, lden, hres, bias_i, Gb):
    bn = 512
    return pl.pallas_call(
        _combine_kernel,
        grid=(N_PAD // bn,),
        in_specs=[
            pl.BlockSpec((NC, bn, D), lambda i: (0, i, 0)),
            pl.BlockSpec((NC, bn, H), lambda i: (0, i, 0)),
            pl.BlockSpec((bn, D), lambda i: (i, 0)),
            pl.BlockSpec((bn, D), lambda i: (i, 0)),
            pl.BlockSpec((bn, D), lambda i: (i, 0)),
            pl.BlockSpec((1, D), lambda i: (0, 0)),
            pl.BlockSpec((H, D), lambda i: (0, 0)),
        ],
        out_specs=pl.BlockSpec((bn, D), lambda i: (i, 0)),
        out_shape=jax.ShapeDtypeStruct((N_PAD, D), jnp.float32),
    )(accn, den8, lnum, lden, hres, bias_i[None, :], Gb)


def _head_kernel(h_ref, w1_ref, b1_ref, w2_ref, b2_ref, w3_ref, b3_ref, o_ref):
    rows = lax.broadcasted_iota(jnp.int32, (N_PAD, 1), 0)
    hm = jnp.where(rows < N, h_ref[...], 0.0)
    g = jnp.sum(hm, axis=0, keepdims=True) * (1.0 / N)
    g = _elu(jnp.dot(g, w1_ref[...], preferred_element_type=jnp.float32) + b1_ref[...])
    g = _elu(jnp.dot(g, w2_ref[...], preferred_element_type=jnp.float32) + b2_ref[...])
    logits = jnp.dot(g, w3_ref[...], preferred_element_type=jnp.float32) + b3_ref[...]
    z = logits - jnp.max(logits, axis=-1, keepdims=True)
    ez = jnp.exp(z)
    o_ref[...] = ez / jnp.sum(ez, axis=-1, keepdims=True)


def _head(h, W1, b1, W2, b2, W3, b3):
    return pl.pallas_call(
        _head_kernel,
        out_shape=jax.ShapeDtypeStruct((1, OUT), jnp.float32),
    )(h, W1, b1[None, :], W2, b2[None, :], W3, b3[None, :])


def kernel(x, edge_index, edge_attr, Wenc, benc, Wee, bee, Wl, bl, Wr, br,
           We, att, bias, Wres, bres, W1, b1, W2, b2, W3, b3):
    src, dst = edge_index[0], edge_index[1]

    pad_e = E_PAD - E
    src_p = jnp.concatenate([src, jnp.full((pad_e,), N, jnp.int32)])
    dst_p = jnp.concatenate([dst, jnp.full((pad_e,), N, jnp.int32)])
    x_pad = jnp.zeros((N_PAD, ND), jnp.float32).at[:N].set(x)
    ea_pad = jnp.zeros((E_PAD, ED), jnp.float32).at[:E].set(edge_attr)
    kk = jnp.arange(D)
    G = (kk[:, None] // C == kk[None, :] // C).astype(jnp.float32)
    Gb = (jnp.arange(H)[:, None] == kk[None, :] // C).astype(jnp.float32)
    att_rows = att.reshape(L, 1, D)
    attflat = att.reshape(L, D)
    zero_acc = jnp.zeros((N_PAD, D), jnp.float32)

    h = _encode_x(x_pad, Wenc, benc)
    ee, ee_sum = _encode_e(ea_pad, Wee, bee)
    mean_ee = ee_sum * (1.0 / E)

    ep_all = _eproj_all(ee, We)

    for i in range(L):
        xl, xr, hres, lnum, lden = _project(
            h, Wl[i], bl[i], Wr[i], br[i], Wres[i], bres[i], We[i],
            mean_ee, att_rows[i], G)
        accn, accd = _sc_edges(xl, xr, ep_all[i], src_p, dst_p, attflat[i],
                               zero_acc)
        den8 = accd.reshape(NC, N_PAD, H)
        h = _combine_call(accn, den8, lnum, lden, hres, bias[i], Gb)

    return _head(h, W1, b1, W2, b2, W3, b3)
